# baseline (speedup 1.0000x reference)
import functools

import jax
import jax.numpy as jnp
from jax import lax
from jax.experimental import pallas as pl
from jax.experimental.pallas import tpu as pltpu
from jax.experimental.pallas import tpu_sc as plsc


_BV = 5000


def _stage_body(l_ref, r_ref, o_ref):
    o_ref[:, : l_ref.shape[1]] = l_ref[...]
    o_ref[:, l_ref.shape[1]:] = r_ref[...]


@functools.lru_cache(maxsize=None)
def _make_stage(vocab, emb):
    grid = pl.cdiv(vocab, _BV)
    return pl.pallas_call(
        _stage_body,
        grid=(grid,),
        in_specs=[
            pl.BlockSpec((_BV, emb), lambda i: (i, 0)),
            pl.BlockSpec((_BV, emb), lambda i: (i, 0)),
        ],
        out_specs=pl.BlockSpec((_BV, 2 * emb), lambda i: (i, 0)),
        out_shape=jax.ShapeDtypeStruct((vocab, 2 * emb), jnp.float32),
    )


_CH = 16


@functools.lru_cache(maxsize=None)
def _make_sc_gather(vocab, emb, batch):
    info = plsc.get_sparse_core_info()
    nc, ns = info.num_cores, info.num_subcores
    nw = nc * ns
    b_per_w = batch // nw
    n_chunks = b_per_w // _CH
    assert batch % nw == 0 and b_per_w % _CH == 0
    mesh = plsc.VectorSubcoreMesh(core_axis_name="c", subcore_axis_name="s")

    @functools.partial(
        pl.kernel,
        out_type=jax.ShapeDtypeStruct((batch, 2 * emb), jnp.float32),
        mesh=mesh,
        scratch_types=[
            pltpu.VMEM((b_per_w,), jnp.int32),
            pltpu.VMEM((b_per_w,), jnp.int32),
            pltpu.VMEM((b_per_w, 2 * emb), jnp.float32),
            pltpu.SemaphoreType.DMA,
        ],
    )
    def sc_gather(ft_hbm, li_hbm, ri_hbm, out_hbm,
                  lidx_v, ridx_v, rows_v, sem):
        wid = lax.axis_index("s") * nc + lax.axis_index("c")
        base = wid * b_per_w
        pltpu.sync_copy(li_hbm.at[wid], lidx_v)
        pltpu.sync_copy(ri_hbm.at[wid], ridx_v)

        def fire(c):
            vl = lidx_v[pl.ds(c * _CH, _CH)]
            vr = ridx_v[pl.ds(c * _CH, _CH)]
            for j in range(_CH):
                i = c * _CH + j
                pltpu.async_copy(ft_hbm.at[vl[j], pl.ds(0, emb)],
                                 rows_v.at[i, pl.ds(0, emb)], sem)
                pltpu.async_copy(ft_hbm.at[vr[j], pl.ds(emb, emb)],
                                 rows_v.at[i, pl.ds(emb, emb)], sem)

        def drain():
            for _ in range(2 * _CH):
                pltpu.make_async_copy(
                    ft_hbm.at[0, pl.ds(0, emb)],
                    rows_v.at[0, pl.ds(0, emb)], sem).wait()

        def body(c, carry):
            fire(c)

            @pl.when(c > 0)
            def _():
                drain()

            return carry

        lax.fori_loop(0, n_chunks, body, 0, unroll=False)
        drain()
        pltpu.sync_copy(rows_v, out_hbm.at[pl.ds(base, b_per_w)])

    return sc_gather


_BM = 2048


def _mm_body(c_ref, wt_ref, b_ref, o_ref):
    o_ref[...] = jnp.dot(c_ref[...], wt_ref[...],
                         preferred_element_type=jnp.float32) + b_ref[...]


@functools.lru_cache(maxsize=None)
def _make_mm(batch, emb2, dim):
    grid = batch // _BM
    return pl.pallas_call(
        _mm_body,
        grid=(grid,),
        in_specs=[
            pl.BlockSpec((_BM, emb2), lambda i: (i, 0)),
            pl.BlockSpec((emb2, dim), lambda i: (0, 0)),
            pl.BlockSpec((1, dim), lambda i: (0, 0)),
        ],
        out_specs=pl.BlockSpec((_BM, dim), lambda i: (i, 0)),
        out_shape=jax.ShapeDtypeStruct((batch, dim), jnp.float32),
    )


def kernel(x, left_table, right_table, W, b):
    batch = x.shape[0]
    vocab, emb = left_table.shape
    dim = W.shape[0]
    info = plsc.get_sparse_core_info()
    nw = info.num_cores * info.num_subcores
    xi = x.astype(jnp.int32)
    left_idx = xi[:, 0].reshape(nw, batch // nw)
    right_idx = xi[:, 1].reshape(nw, batch // nw)
    fused = _make_stage(vocab, emb)(left_table, right_table)
    concat_enc = _make_sc_gather(vocab, emb, batch)(
        fused, left_idx, right_idx)
    return _make_mm(batch, 2 * emb, dim)(concat_enc, W.T, b.reshape(1, dim))


# speedup vs baseline: 1.3810x; 1.3810x over previous
import functools

import jax
import jax.numpy as jnp
from jax import lax
from jax.experimental import pallas as pl
from jax.experimental.pallas import tpu as pltpu
from jax.experimental.pallas import tpu_sc as plsc


_CH = 16


@functools.lru_cache(maxsize=None)
def _make_sc_gather(vocab, emb, batch):
    info = plsc.get_sparse_core_info()
    nc, ns = info.num_cores, info.num_subcores
    nw = nc * ns
    b_per_w = batch // nw
    n_chunks = b_per_w // _CH
    assert batch % nw == 0 and b_per_w % _CH == 0
    mesh = plsc.VectorSubcoreMesh(core_axis_name="c", subcore_axis_name="s")

    @functools.partial(
        pl.kernel,
        out_type=jax.ShapeDtypeStruct((batch, emb), jnp.float32),
        mesh=mesh,
        scratch_types=[
            pltpu.VMEM((b_per_w,), jnp.int32),
            pltpu.VMEM((b_per_w, emb), jnp.float32),
            pltpu.SemaphoreType.DMA,
        ],
    )
    def sc_gather(tbl_hbm, idx_hbm, out_hbm, idx_v, rows_v, sem):
        wid = lax.axis_index("s") * nc + lax.axis_index("c")
        base = wid * b_per_w
        pltpu.sync_copy(idx_hbm.at[wid], idx_v)

        def fire(c):
            vv = idx_v[pl.ds(c * _CH, _CH)]
            for j in range(_CH):
                i = c * _CH + j
                pltpu.async_copy(tbl_hbm.at[vv[j]], rows_v.at[i], sem)

        def drain():
            for _ in range(_CH):
                pltpu.make_async_copy(
                    tbl_hbm.at[0], rows_v.at[0], sem).wait()

        def body(c, carry):
            fire(c)

            @pl.when(c > 0)
            def _():
                drain()

            return carry

        lax.fori_loop(0, n_chunks, body, 0, unroll=False)
        drain()
        pltpu.sync_copy(rows_v, out_hbm.at[pl.ds(base, b_per_w)])

    return sc_gather


_BM = 2048


def _mm_body(l_ref, r_ref, wl_ref, wr_ref, b_ref, o_ref):
    acc = jnp.dot(l_ref[...], wl_ref[...], preferred_element_type=jnp.float32)
    acc += jnp.dot(r_ref[...], wr_ref[...], preferred_element_type=jnp.float32)
    o_ref[...] = acc + b_ref[...]


@functools.lru_cache(maxsize=None)
def _make_mm(batch, emb, dim):
    grid = batch // _BM
    return pl.pallas_call(
        _mm_body,
        grid=(grid,),
        in_specs=[
            pl.BlockSpec((_BM, emb), lambda i: (i, 0)),
            pl.BlockSpec((_BM, emb), lambda i: (i, 0)),
            pl.BlockSpec((emb, dim), lambda i: (0, 0)),
            pl.BlockSpec((emb, dim), lambda i: (0, 0)),
            pl.BlockSpec((1, dim), lambda i: (0, 0)),
        ],
        out_specs=pl.BlockSpec((_BM, dim), lambda i: (i, 0)),
        out_shape=jax.ShapeDtypeStruct((batch, dim), jnp.float32),
    )


def kernel(x, left_table, right_table, W, b):
    batch = x.shape[0]
    vocab, emb = left_table.shape
    dim = W.shape[0]
    info = plsc.get_sparse_core_info()
    nw = info.num_cores * info.num_subcores
    xi = x.astype(jnp.int32)
    left_idx = xi[:, 0].reshape(nw, batch // nw)
    right_idx = xi[:, 1].reshape(nw, batch // nw)
    gf = _make_sc_gather(vocab, emb, batch)
    left_enc = gf(left_table, left_idx)
    right_enc = gf(right_table, right_idx)
    wl = W[:, :emb].T
    wr = W[:, emb:].T
    return _make_mm(batch, emb, dim)(
        left_enc, right_enc, wl, wr, b.reshape(1, dim))
